# Initial kernel scaffold; baseline (speedup 1.0000x reference)
import jax
import jax.numpy as jnp
from jax.experimental import pallas as pl


def kernel(x, edge_index, W_l, b_l, W_r, W_res, b_res):
    raise NotImplementedError("write your pallas kernel here")


# speedup vs baseline: 4.3970x; 4.3970x over previous
import functools

import jax
import jax.numpy as jnp
from jax import lax
from jax.experimental import pallas as pl
from jax.experimental.pallas import tpu as pltpu, tpu_sc as plsc

N_NODES = 10000
N_EDGES = 320000
D = 128
RW = 0.001

NC = 2
NS = 16
NW = NC * NS

CH = 128
NCHUNK = 2528
CHUNKS_PER_W = NCHUNK // NW
E_PAD = NCHUNK * CH
N_PAD = 10240
ROWS_PER_SUBCORE = N_PAD // NS
ZROWS = 128


def _sc_body(x_hbm, src_hbm, dst_hbm, part_hbm, agg, idx_s, idx_d, rows, zbuf, sem):
    cid = lax.axis_index("c")
    sid = lax.axis_index("s")
    wid = sid * NC + cid

    def zrow(i, _):
        for j in range(D // 16):
            zbuf[i, pl.ds(j * 16, 16)] = jnp.zeros((16,), jnp.float32)
        return 0

    lax.fori_loop(0, ZROWS, zrow, 0)

    zbase = sid * ROWS_PER_SUBCORE
    for k in range(ROWS_PER_SUBCORE // ZROWS):
        pltpu.sync_copy(zbuf, agg.at[pl.ds(zbase + k * ZROWS, ZROWS)])
    plsc.subcore_barrier()

    cbase = wid * CHUNKS_PER_W

    def chunk(j, _):
        c = cbase + j
        pltpu.sync_copy(src_hbm.at[c], idx_s)
        pltpu.sync_copy(dst_hbm.at[c], idx_d)
        pltpu.async_copy(x_hbm.at[idx_s], rows, sem).wait()
        pltpu.sync_copy(rows, agg.at[idx_d], add=True)
        return 0

    lax.fori_loop(0, CHUNKS_PER_W, chunk, 0)
    plsc.subcore_barrier()

    pltpu.sync_copy(
        agg.at[pl.ds(zbase, ROWS_PER_SUBCORE)],
        part_hbm.at[cid, pl.ds(zbase, ROWS_PER_SUBCORE), :],
    )


_sc_segsum = pl.kernel(
    _sc_body,
    out_type=jax.ShapeDtypeStruct((NC, N_PAD, D), jnp.float32),
    mesh=plsc.VectorSubcoreMesh(
        core_axis_name="c", subcore_axis_name="s", num_cores=NC, num_subcores=NS
    ),
    scratch_types=[
        pltpu.VMEM_SHARED((N_PAD, D), jnp.float32),
        pltpu.VMEM((CH,), jnp.int32),
        pltpu.VMEM((CH,), jnp.int32),
        pltpu.VMEM((CH, D), jnp.float32),
        pltpu.VMEM((ZROWS, D), jnp.float32),
        pltpu.SemaphoreType.DMA,
    ],
)


ROWS_TC = 1000


def _tc_body(p_ref, x_ref, wl_ref, wc_ref, b_ref, o_ref):
    agg = p_ref[0] + p_ref[1]
    o_ref[...] = (
        jnp.dot(agg, wl_ref[...], preferred_element_type=jnp.float32)
        + jnp.dot(x_ref[...], wc_ref[...], preferred_element_type=jnp.float32)
        + b_ref[...]
    )


_tc_fused = pl.pallas_call(
    _tc_body,
    grid=(N_NODES // ROWS_TC,),
    in_specs=[
        pl.BlockSpec((NC, ROWS_TC, D), lambda i: (0, i, 0)),
        pl.BlockSpec((ROWS_TC, D), lambda i: (i, 0)),
        pl.BlockSpec((D, D), lambda i: (0, 0)),
        pl.BlockSpec((D, D), lambda i: (0, 0)),
        pl.BlockSpec((1, D), lambda i: (0, 0)),
    ],
    out_specs=pl.BlockSpec((ROWS_TC, D), lambda i: (i, 0)),
    out_shape=jax.ShapeDtypeStruct((N_NODES, D), jnp.float32),
)


def kernel(x, edge_index, W_l, b_l, W_r, W_res, b_res):
    src = edge_index[0]
    dst = edge_index[1]
    pad = E_PAD - N_EDGES
    src_p = jnp.concatenate([src, jnp.zeros((pad,), jnp.int32)]).reshape(NCHUNK, CH)
    dst_p = jnp.concatenate([dst, jnp.full((pad,), N_NODES, jnp.int32)]).reshape(
        NCHUNK, CH
    )
    partials = _sc_segsum(x, src_p, dst_p)

    wl_t = (RW * W_l).T
    wc_t = (RW * W_r + (1.0 - RW) * W_res).T
    bc = (RW * b_l + (1.0 - RW) * b_res).reshape(1, D)
    return _tc_fused(partials, x, wl_t, wc_t, bc)
